# baseline (speedup 1.0000x reference)
import functools
import jax
import jax.numpy as jnp
from jax import lax
from jax.experimental import pallas as pl
from jax.experimental.pallas import tpu as pltpu
from jax.experimental.pallas import tpu_sc as plsc

B, N, L, D = 8, 9, 512, 768
S = N - 1
NC, NS = 2, 16
NW = NC * NS
WPB = NW // B
RPW = S * L // WPB

_mesh = plsc.VectorSubcoreMesh(core_axis_name="c", subcore_axis_name="s")


@functools.partial(
    pl.kernel,
    out_type=[
        jax.ShapeDtypeStruct((B * S * L, D), jnp.float32),
        jax.ShapeDtypeStruct((B * S, 1, L), jnp.float32),
    ],
    mesh=_mesh,
)
def _sc_select(reps_hbm, mask_hbm, reps_out, mask_out):
    c = lax.axis_index("c")
    s = lax.axis_index("s")
    wid = s * NC + c
    b = wid // WPB
    q = wid % WPB
    in_row = b * (N * L) + L + q * RPW
    out_row = b * (S * L) + q * RPW

    pltpu.sync_copy(reps_hbm.at[pl.ds(in_row, RPW)],
                    reps_out.at[pl.ds(out_row, RPW)])

    @pl.when(q == 0)
    def _copy_mask():
        pltpu.sync_copy(mask_hbm.at[pl.ds(b * N + 1, S)],
                        mask_out.at[pl.ds(b * S, S)])


def kernel(token_reps, token_mask, valid_sentences):
    reps2d = token_reps.reshape(B * N * L, D)
    mask2d = token_mask.reshape(B * N, 1, L)
    reps_out, mask_out = _sc_select(reps2d, mask2d)
    return reps_out.reshape(B, S, L, D), mask_out.reshape(B, S, L)


# speedup vs baseline: 32.9628x; 32.9628x over previous
import functools
import jax
import jax.numpy as jnp
from jax import lax
from jax.experimental import pallas as pl
from jax.experimental.pallas import tpu as pltpu
from jax.experimental.pallas import tpu_sc as plsc

B, N, L, D = 8, 9, 512, 768
S = N - 1
NC, NS = 2, 16
NW = NC * NS
WPB = NW // B
RPW = S * L // WPB
CH = 64
NCH = RPW // CH

_mesh = plsc.VectorSubcoreMesh(core_axis_name="c", subcore_axis_name="s")


@functools.partial(
    pl.kernel,
    out_type=[
        jax.ShapeDtypeStruct((B * S * L, D), jnp.float32),
        jax.ShapeDtypeStruct((B * S, 1, L), jnp.float32),
    ],
    mesh=_mesh,
    scratch_types=[
        pltpu.MemorySpace.VMEM((CH, D), jnp.float32),
        pltpu.MemorySpace.VMEM((CH, D), jnp.float32),
        pltpu.SemaphoreType.DMA,
        pltpu.SemaphoreType.DMA,
        pltpu.SemaphoreType.DMA,
        pltpu.SemaphoreType.DMA,
    ],
)
def _sc_select(reps_hbm, mask_hbm, reps_out, mask_out,
               buf0, buf1, sin0, sin1, sout0, sout1):
    c = lax.axis_index("c")
    s = lax.axis_index("s")
    wid = s * NC + c
    b = wid // WPB
    q = wid % WPB
    in_row = b * (N * L) + L + q * RPW
    out_row = b * (S * L) + q * RPW

    bufs = (buf0, buf1)
    sins = (sin0, sin1)
    souts = (sout0, sout1)
    mk = pltpu.make_async_copy

    def in_cp(k):
        return mk(reps_hbm.at[pl.ds(in_row + k * CH, CH)], bufs[k % 2],
                  sins[k % 2])

    def out_cp(k):
        return mk(bufs[k % 2], reps_out.at[pl.ds(out_row + k * CH, CH)],
                  souts[k % 2])

    in_cp(0).start()
    for k in range(NCH):
        if k + 1 < NCH:
            if k >= 1:
                out_cp(k - 1).wait()
            in_cp(k + 1).start()
        in_cp(k).wait()
        out_cp(k).start()
    out_cp(NCH - 2).wait()
    out_cp(NCH - 1).wait()

    @pl.when(q == 0)
    def _copy_mask():
        pltpu.sync_copy(mask_hbm.at[pl.ds(b * N + 1, S)],
                        mask_out.at[pl.ds(b * S, S)])


def kernel(token_reps, token_mask, valid_sentences):
    reps2d = token_reps.reshape(B * N * L, D)
    mask2d = token_mask.reshape(B * N, 1, L)
    reps_out, mask_out = _sc_select(reps2d, mask2d)
    return reps_out.reshape(B, S, L, D), mask_out.reshape(B, S, L)
